# baseline (speedup 1.0000x reference)
import functools

import jax
import jax.numpy as jnp
from jax import lax
from jax.experimental import pallas as pl
from jax.experimental.pallas import tpu as pltpu
from jax.experimental.pallas import tpu_sc as plsc

_IN = 4096
_OUT = 4096
_B = 64
_NNZ = 167772

_NC = 2
_NS = 16
_NW = _NC * _NS
_K = 128
_NMAIN = _NNZ // (_NW * _K)
_NCHUNK = _NMAIN + 1
_NQ4 = (_NCHUNK - 5) // 4
_NTAIL = _NNZ - _NW * _NMAIN * _K
_ROWS_PER_TILE = _OUT // _NS


def _sc_body(xt_hbm, col_hbm, row_hbm, w_hbm, colt_hbm, rowt_hbm, wt_hbm,
             out_hbm, colv, rowv, wv, gbuf0, gbuf1, gbuf2, gbuf3,
             sbuf0, sbuf1, sbuf2, sbuf3, acc,
             gsem0, gsem1, gsem2, gsem3, ssem0, ssem1, ssem2, ssem3):
    c = lax.axis_index("c")
    s = lax.axis_index("s")
    wid = c * _NS + s

    pltpu.sync_copy(col_hbm.at[wid], colv.at[pl.ds(0, _NMAIN)])
    pltpu.sync_copy(row_hbm.at[wid], rowv.at[pl.ds(0, _NMAIN)])
    pltpu.sync_copy(w_hbm.at[wid], wv.at[pl.ds(0, _NMAIN * _K)])
    pltpu.sync_copy(colt_hbm.at[wid], colv.at[_NMAIN])
    pltpu.sync_copy(rowt_hbm.at[wid], rowv.at[_NMAIN])
    pltpu.sync_copy(wt_hbm.at[wid], wv.at[pl.ds(_NMAIN * _K, _K)])

    zeros = jnp.zeros((16,), jnp.float32)

    def _zero_row(r, carry):
        for j in range(_B // 16):
            sbuf0[r, pl.ds(16 * j, 16)] = zeros
        return carry

    lax.fori_loop(0, _K, _zero_row, 0)
    pltpu.sync_copy(sbuf0, acc.at[pl.ds(s * _ROWS_PER_TILE, _K)])
    pltpu.sync_copy(sbuf0, acc.at[pl.ds(s * _ROWS_PER_TILE + _K, _K)])
    plsc.subcore_barrier()

    def _scale(i, src, dst):
        @plsc.parallel_loop(0, _K // 16, unroll=2)
        def _group(g):
            base = g * 16
            wvec = wv[pl.ds(i * _K + base, 16)]
            for l in range(16):
                w = wvec[l]
                for j in range(_B // 16):
                    sl = pl.ds(16 * j, 16)
                    dst[base + l, sl] = src[base + l, sl] * w

    gb = (gbuf0, gbuf1, gbuf2, gbuf3)
    sb = (sbuf0, sbuf1, sbuf2, sbuf3)
    gs = (gsem0, gsem1, gsem2, gsem3)
    ss = (ssem0, ssem1, ssem2, ssem3)

    def _gather(i, q):
        return pltpu.make_async_copy(xt_hbm.at[colv.at[i]], gb[q], gs[q])

    def _scat(i, q):
        return pltpu.make_async_copy(sb[q], acc.at[rowv.at[i]], ss[q])

    _gather(0, 0).start()
    _gather(1, 1).start()
    _gather(2, 2).start()

    def _quad(p, carry):
        c = 4 * p
        for q in range(4):
            _gather(c + q, q).wait()
            _gather(c + q + 3, (q + 3) % 4).start()

            @pl.when(p > 0)
            def _():
                _scat(c + q - 4, q).wait()

            _scale(c + q, gb[q], sb[q])
            _scat(c + q, q).start(add=True)
        return carry

    lax.fori_loop(0, _NQ4, _quad, 0)

    e = 4 * _NQ4
    _gather(e, 0).wait()
    _scat(e - 4, 0).wait()
    _scale(e, gbuf0, sbuf0)
    _scat(e, 0).start(add=True)
    _gather(e + 3, 3).start()

    _gather(e + 1, 1).wait()
    _scat(e - 3, 1).wait()
    _scale(e + 1, gbuf1, sbuf1)
    _scat(e + 1, 1).start(add=True)
    _gather(e + 4, 0).start()

    _gather(e + 2, 2).wait()
    _scat(e - 2, 2).wait()
    _scale(e + 2, gbuf2, sbuf2)
    _scat(e + 2, 2).start(add=True)

    _gather(e + 3, 3).wait()
    _scat(e - 1, 3).wait()
    _scale(e + 3, gbuf3, sbuf3)
    _scat(e + 3, 3).start(add=True)

    _gather(e + 4, 0).wait()
    _scat(e, 0).wait()
    _scale(e + 4, gbuf0, sbuf0)
    _scat(e + 4, 0).start(add=True)

    _scat(e + 4, 0).wait()
    _scat(e + 1, 1).wait()
    _scat(e + 2, 2).wait()
    _scat(e + 3, 3).wait()
    plsc.subcore_barrier()

    base = s * _ROWS_PER_TILE
    pltpu.sync_copy(acc.at[pl.ds(base, _ROWS_PER_TILE)],
                    out_hbm.at[c, pl.ds(base, _ROWS_PER_TILE)])


@jax.jit
def _sparse_linear(xt, colp, rowp, wp, colt, rowt, wt):
    mesh = plsc.VectorSubcoreMesh(core_axis_name="c", subcore_axis_name="s")
    run = pl.kernel(
        _sc_body,
        out_type=jax.ShapeDtypeStruct((_NC, _OUT, _B), jnp.float32),
        mesh=mesh,
        compiler_params=pltpu.CompilerParams(use_tc_tiling_on_sc=False),
        scratch_types=[
            pltpu.VMEM((_NCHUNK, _K), jnp.int32),
            pltpu.VMEM((_NCHUNK, _K), jnp.int32),
            pltpu.VMEM((_NCHUNK * _K,), jnp.float32),
            pltpu.VMEM((_K, _B), jnp.float32),
            pltpu.VMEM((_K, _B), jnp.float32),
            pltpu.VMEM((_K, _B), jnp.float32),
            pltpu.VMEM((_K, _B), jnp.float32),
            pltpu.VMEM((_K, _B), jnp.float32),
            pltpu.VMEM((_K, _B), jnp.float32),
            pltpu.VMEM((_K, _B), jnp.float32),
            pltpu.VMEM((_K, _B), jnp.float32),
            pltpu.VMEM_SHARED((_OUT, _B), jnp.float32),
            pltpu.SemaphoreType.DMA,
            pltpu.SemaphoreType.DMA,
            pltpu.SemaphoreType.DMA,
            pltpu.SemaphoreType.DMA,
            pltpu.SemaphoreType.DMA,
            pltpu.SemaphoreType.DMA,
            pltpu.SemaphoreType.DMA,
            pltpu.SemaphoreType.DMA,
        ],
    )
    return run(xt, colp, rowp, wp, colt, rowt, wt)


def kernel(inputs, weights, row, col):
    x = inputs.reshape(-1, _IN)
    xt = x.T

    n = _NW * _NMAIN * _K
    colp = col[:n].reshape(_NW, _NMAIN, _K)
    rowp = row[:n].reshape(_NW, _NMAIN, _K)
    wp = weights[:n].reshape(_NW, _NMAIN * _K)
    padt = _NW * _K - _NTAIL
    zi = jnp.zeros((padt,), jnp.int32)
    colt = jnp.concatenate([col[n:], zi]).reshape(_NW, _K)
    rowt = jnp.concatenate([row[n:], zi]).reshape(_NW, _K)
    wt = jnp.concatenate([weights[n:], jnp.zeros((padt,), jnp.float32)]
                         ).reshape(_NW, _K)

    part = _sparse_linear(xt, colp, rowp, wp, colt, rowt, wt)
    y = (part[0] + part[1]).T
    return y.reshape(*inputs.shape[:-1], _OUT)


# speedup vs baseline: 1.0408x; 1.0408x over previous
import functools

import jax
import jax.numpy as jnp
from jax import lax
from jax.experimental import pallas as pl
from jax.experimental.pallas import tpu as pltpu
from jax.experimental.pallas import tpu_sc as plsc

_IN = 4096
_OUT = 4096
_B = 64
_NNZ = 167772

_NC = 2
_NS = 16
_NW = _NC * _NS
_K = 128
_NMAIN = _NNZ // (_NW * _K)
_NCHUNK = _NMAIN + 1
_NT3 = (_NCHUNK - 2) // 3
_NTAIL = _NNZ - _NW * _NMAIN * _K
_ROWS_PER_TILE = _OUT // _NS


def _sc_body(xt_hbm, col_hbm, row_hbm, w_hbm, colt_hbm, rowt_hbm, wt_hbm,
             out_hbm, colv, rowv, wv, gbuf0, gbuf1, gbuf2,
             sbuf0, sbuf1, sbuf2, acc,
             gsem0, gsem1, gsem2, ssem0, ssem1, ssem2):
    c = lax.axis_index("c")
    s = lax.axis_index("s")
    wid = c * _NS + s

    nm = _NMAIN * _K
    pltpu.sync_copy(col_hbm.at[pl.ds(wid * nm, nm)], colv.at[pl.ds(0, nm)])
    pltpu.sync_copy(row_hbm.at[wid], rowv.at[pl.ds(0, _NMAIN)])
    pltpu.sync_copy(w_hbm.at[pl.ds(wid * nm, nm)], wv.at[pl.ds(0, nm)])
    pltpu.sync_copy(colt_hbm.at[wid], colv.at[pl.ds(nm, _K)])
    pltpu.sync_copy(rowt_hbm.at[wid], rowv.at[_NMAIN])
    pltpu.sync_copy(wt_hbm.at[wid], wv.at[pl.ds(nm, _K)])

    zeros = jnp.zeros((16,), jnp.float32)

    def _zero_row(r, carry):
        for j in range(_B // 16):
            sbuf0[r, pl.ds(16 * j, 16)] = zeros
        return carry

    lax.fori_loop(0, _K, _zero_row, 0)
    pltpu.sync_copy(sbuf0, acc.at[pl.ds(s * _ROWS_PER_TILE, _K)])
    pltpu.sync_copy(sbuf0, acc.at[pl.ds(s * _ROWS_PER_TILE + _K, _K)])
    plsc.subcore_barrier()

    def _scale(i, src, dst):
        @plsc.parallel_loop(0, _K // 16, unroll=2)
        def _group(g):
            base = g * 16
            wvec = wv[pl.ds(i * _K + base, 16)]
            for l in range(16):
                w = wvec[l]
                for j in range(_B // 16):
                    sl = pl.ds(16 * j, 16)
                    dst[base + l, sl] = src[base + l, sl] * w

    gb = (gbuf0, gbuf1, gbuf2)
    sb = (sbuf0, sbuf1, sbuf2)
    gs = (gsem0, gsem1, gsem2)
    ss = (ssem0, ssem1, ssem2)

    def _gather(i, q):
        return pltpu.make_async_copy(
            xt_hbm.at[colv.at[pl.ds(i * _K, _K)]], gb[q], gs[q])

    def _scat(i, q):
        return pltpu.make_async_copy(sb[q], acc.at[rowv.at[i]], ss[q])

    _gather(0, 0).start()
    _gather(1, 1).start()

    def _triple(p, carry):
        c = 3 * p
        for q in range(3):
            _gather(c + q, q).wait()
            _gather(c + q + 2, (q + 2) % 3).start()

            @pl.when(p > 0)
            def _():
                _scat(c + q - 3, q).wait()

            _scale(c + q, gb[q], sb[q])
            _scat(c + q, q).start(add=True)
        return carry

    lax.fori_loop(0, _NT3, _triple, 0)

    c39 = 3 * _NT3
    _gather(c39, 0).wait()
    _scat(c39 - 3, 0).wait()
    _scale(c39, gbuf0, sbuf0)
    _scat(c39, 0).start(add=True)
    _gather(c39 + 1, 1).wait()
    _scat(c39 - 2, 1).wait()
    _scale(c39 + 1, gbuf1, sbuf1)
    _scat(c39 + 1, 1).start(add=True)
    _scat(c39, 0).wait()
    _scat(c39 + 1, 1).wait()
    _scat(c39 - 1, 2).wait()
    plsc.subcore_barrier()

    base = s * _ROWS_PER_TILE
    pltpu.sync_copy(acc.at[pl.ds(base, _ROWS_PER_TILE)],
                    out_hbm.at[c, pl.ds(base, _ROWS_PER_TILE)])


@jax.jit
def _sparse_linear(xt, colp, rowp, wp, colt, rowt, wt):
    mesh = plsc.VectorSubcoreMesh(core_axis_name="c", subcore_axis_name="s")
    run = pl.kernel(
        _sc_body,
        out_type=jax.ShapeDtypeStruct((_NC, _OUT, _B), jnp.float32),
        mesh=mesh,
        compiler_params=pltpu.CompilerParams(use_tc_tiling_on_sc=False),
        scratch_types=[
            pltpu.VMEM((_NCHUNK * _K,), jnp.int32),
            pltpu.VMEM((_NCHUNK, _K), jnp.int32),
            pltpu.VMEM((_NCHUNK * _K,), jnp.float32),
            pltpu.VMEM((_K, _B), jnp.float32),
            pltpu.VMEM((_K, _B), jnp.float32),
            pltpu.VMEM((_K, _B), jnp.float32),
            pltpu.VMEM((_K, _B), jnp.float32),
            pltpu.VMEM((_K, _B), jnp.float32),
            pltpu.VMEM((_K, _B), jnp.float32),
            pltpu.VMEM_SHARED((_OUT, _B), jnp.float32),
            pltpu.SemaphoreType.DMA,
            pltpu.SemaphoreType.DMA,
            pltpu.SemaphoreType.DMA,
            pltpu.SemaphoreType.DMA,
            pltpu.SemaphoreType.DMA,
            pltpu.SemaphoreType.DMA,
        ],
    )
    return run(xt, colp, rowp, wp, colt, rowt, wt)


def kernel(inputs, weights, row, col):
    x = inputs.reshape(-1, _IN)
    xt = x.T

    n = _NW * _NMAIN * _K
    rowp = row[:n].reshape(_NW, _NMAIN, _K)
    padt = _NW * _K - _NTAIL
    zi = jnp.zeros((padt,), jnp.int32)
    colt = jnp.concatenate([col[n:], zi]).reshape(_NW, _K)
    rowt = jnp.concatenate([row[n:], zi]).reshape(_NW, _K)
    wt = jnp.concatenate([weights[n:], jnp.zeros((padt,), jnp.float32)]
                         ).reshape(_NW, _K)

    part = _sparse_linear(xt, col, rowp, weights, colt, rowt, wt)
    y = (part[0] + part[1]).T
    return y.reshape(*inputs.shape[:-1], _OUT)


# speedup vs baseline: 1.0511x; 1.0099x over previous
import functools

import jax
import jax.numpy as jnp
from jax import lax
from jax.experimental import pallas as pl
from jax.experimental.pallas import tpu as pltpu
from jax.experimental.pallas import tpu_sc as plsc

_IN = 4096
_OUT = 4096
_B = 64
_NNZ = 167772

_NC = 2
_NS = 16
_NW = _NC * _NS
_K = 128
_NMAIN = _NNZ // (_NW * _K)
_NCHUNK = _NMAIN + 1
_NT3 = (_NCHUNK - 2) // 3
_NTAIL = _NNZ - _NW * _NMAIN * _K
_ROWS_PER_TILE = _OUT // _NS


def _sc_body(xt_hbm, col_hbm, row_hbm, w_hbm, colt_hbm, rowt_hbm, wt_hbm,
             out_hbm, colv, rowv, wv, gbuf0, gbuf1, gbuf2,
             sbuf0, sbuf1, sbuf2, acc,
             gsem0, gsem1, gsem2, ssem0, ssem1, ssem2):
    c = lax.axis_index("c")
    s = lax.axis_index("s")
    wid = c * _NS + s

    nm = _NMAIN * _K
    pltpu.sync_copy(col_hbm.at[pl.ds(wid * nm, nm)], colv.at[pl.ds(0, nm)])
    pltpu.sync_copy(row_hbm.at[pl.ds(wid * nm, nm)], rowv.at[pl.ds(0, nm)])
    pltpu.sync_copy(w_hbm.at[pl.ds(wid * nm, nm)], wv.at[pl.ds(0, nm)])
    pltpu.sync_copy(colt_hbm.at[wid], colv.at[pl.ds(nm, _K)])
    pltpu.sync_copy(rowt_hbm.at[wid], rowv.at[pl.ds(nm, _K)])
    pltpu.sync_copy(wt_hbm.at[wid], wv.at[pl.ds(nm, _K)])

    zeros = jnp.zeros((16,), jnp.float32)

    def _zero_row(r, carry):
        for j in range(_B // 16):
            sbuf0[r, pl.ds(16 * j, 16)] = zeros
        return carry

    lax.fori_loop(0, _K, _zero_row, 0)
    pltpu.sync_copy(sbuf0, acc.at[pl.ds(s * _ROWS_PER_TILE, _K)])
    pltpu.sync_copy(sbuf0, acc.at[pl.ds(s * _ROWS_PER_TILE + _K, _K)])
    plsc.subcore_barrier()

    def _scale(i, src, dst):
        @plsc.parallel_loop(0, _K // 16, unroll=2)
        def _group(g):
            base = g * 16
            wvec = wv[pl.ds(i * _K + base, 16)]
            for l in range(16):
                w = wvec[l]
                for j in range(_B // 16):
                    sl = pl.ds(16 * j, 16)
                    dst[base + l, sl] = src[base + l, sl] * w

    gb = (gbuf0, gbuf1, gbuf2)
    sb = (sbuf0, sbuf1, sbuf2)
    gs = (gsem0, gsem1, gsem2)
    ss = (ssem0, ssem1, ssem2)

    def _gather(i, q):
        return pltpu.make_async_copy(
            xt_hbm.at[colv.at[pl.ds(i * _K, _K)]], gb[q], gs[q])

    def _scat(i, q):
        return pltpu.make_async_copy(
            sb[q], acc.at[rowv.at[pl.ds(i * _K, _K)]], ss[q])

    _gather(0, 0).start()
    _gather(1, 1).start()

    def _triple(p, carry):
        c = 3 * p
        for q in range(3):
            _gather(c + q, q).wait()
            _gather(c + q + 2, (q + 2) % 3).start()

            @pl.when(p > 0)
            def _():
                _scat(c + q - 3, q).wait()

            _scale(c + q, gb[q], sb[q])
            _scat(c + q, q).start(add=True)
        return carry

    lax.fori_loop(0, _NT3, _triple, 0)

    c39 = 3 * _NT3
    _gather(c39, 0).wait()
    _scat(c39 - 3, 0).wait()
    _scale(c39, gbuf0, sbuf0)
    _scat(c39, 0).start(add=True)
    _gather(c39 + 1, 1).wait()
    _scat(c39 - 2, 1).wait()
    _scale(c39 + 1, gbuf1, sbuf1)
    _scat(c39 + 1, 1).start(add=True)
    _scat(c39, 0).wait()
    _scat(c39 + 1, 1).wait()
    _scat(c39 - 1, 2).wait()
    plsc.subcore_barrier()

    base = s * _ROWS_PER_TILE
    pltpu.sync_copy(acc.at[pl.ds(base, _ROWS_PER_TILE)],
                    out_hbm.at[c, pl.ds(base, _ROWS_PER_TILE)])


@jax.jit
def _sparse_linear(xt, colp, rowp, wp, colt, rowt, wt):
    mesh = plsc.VectorSubcoreMesh(core_axis_name="c", subcore_axis_name="s")
    run = pl.kernel(
        _sc_body,
        out_type=jax.ShapeDtypeStruct((_NC, _OUT, _B), jnp.float32),
        mesh=mesh,
        compiler_params=pltpu.CompilerParams(use_tc_tiling_on_sc=False),
        scratch_types=[
            pltpu.VMEM((_NCHUNK * _K,), jnp.int32),
            pltpu.VMEM((_NCHUNK * _K,), jnp.int32),
            pltpu.VMEM((_NCHUNK * _K,), jnp.float32),
            pltpu.VMEM((_K, _B), jnp.float32),
            pltpu.VMEM((_K, _B), jnp.float32),
            pltpu.VMEM((_K, _B), jnp.float32),
            pltpu.VMEM((_K, _B), jnp.float32),
            pltpu.VMEM((_K, _B), jnp.float32),
            pltpu.VMEM((_K, _B), jnp.float32),
            pltpu.VMEM_SHARED((_OUT, _B), jnp.float32),
            pltpu.SemaphoreType.DMA,
            pltpu.SemaphoreType.DMA,
            pltpu.SemaphoreType.DMA,
            pltpu.SemaphoreType.DMA,
            pltpu.SemaphoreType.DMA,
            pltpu.SemaphoreType.DMA,
        ],
    )
    return run(xt, colp, rowp, wp, colt, rowt, wt)


def kernel(inputs, weights, row, col):
    x = inputs.reshape(-1, _IN)
    xt = x.T

    n = _NW * _NMAIN * _K
    padt = _NW * _K - _NTAIL
    zi = jnp.zeros((padt,), jnp.int32)
    colt = jnp.concatenate([col[n:], zi]).reshape(_NW, _K)
    rowt = jnp.concatenate([row[n:], zi]).reshape(_NW, _K)
    wt = jnp.concatenate([weights[n:], jnp.zeros((padt,), jnp.float32)]
                         ).reshape(_NW, _K)

    part = _sparse_linear(xt, col, row, weights, colt, rowt, wt)
    y = (part[0] + part[1]).T
    return y.reshape(*inputs.shape[:-1], _OUT)
